# Initial kernel scaffold; baseline (speedup 1.0000x reference)
import jax
import jax.numpy as jnp
from jax.experimental import pallas as pl


def kernel(memory, veh_idx, veh_repr, cust_repr, edge_emb, W_in, b_in, W_h, b_h, ln_gamma, ln_beta):
    raise NotImplementedError("write your pallas kernel here")


# speedup vs baseline: 112.8276x; 112.8276x over previous
import jax
import jax.numpy as jnp
from jax.experimental import pallas as pl

BLOCK_N = 256


def _fused_body(mem_ref, idx_ref, x_ref, w_in_t_ref, w_h_t_ref, bias_ref,
                gamma_ref, beta_ref, out_ref):
    b, l, h = mem_ref.shape
    mem = mem_ref[...]
    idx3 = jax.lax.broadcast_in_dim(idx_ref[...], (b, l, h), (0, 2))
    lane3 = jax.lax.broadcasted_iota(jnp.int32, (b, l, h), 1)
    sel = lane3 == idx3
    cur_h = jnp.sum(jnp.where(sel, mem, 0.0), axis=1)

    x = x_ref[...]
    pre = (jnp.dot(x, w_in_t_ref[...], preferred_element_type=jnp.float32)
           + jnp.dot(cur_h, w_h_t_ref[...], preferred_element_type=jnp.float32)
           + bias_ref[...])
    mean = jnp.mean(pre, axis=-1, keepdims=True)
    cent = pre - mean
    var = jnp.mean(cent * cent, axis=-1, keepdims=True)
    normed = cent * jax.lax.rsqrt(var + 1e-5) * gamma_ref[...] + beta_ref[...]
    next_h = jnp.tanh(normed)

    nh3 = jax.lax.broadcast_in_dim(next_h, (b, l, h), (0, 2))
    out_ref[...] = jnp.where(sel, nh3, mem)


def kernel(memory, veh_idx, veh_repr, cust_repr, edge_emb,
           W_in, b_in, W_h, b_h, ln_gamma, ln_beta):
    n, l, h = memory.shape
    d = veh_repr.shape[-1]
    x = jnp.concatenate(
        [veh_repr[:, 0, :], cust_repr[:, 0, :], edge_emb[:, 0, 0, :]], axis=-1)
    w_in_t = W_in.T
    w_h_t = W_h.T
    bias = (b_in + b_h).reshape(1, h)
    gamma = ln_gamma.reshape(1, h)
    beta = ln_beta.reshape(1, h)
    idx = jnp.broadcast_to(veh_idx.astype(jnp.int32), (n, h))

    grid = (n // BLOCK_N,)
    return pl.pallas_call(
        _fused_body,
        grid=grid,
        in_specs=[
            pl.BlockSpec((BLOCK_N, l, h), lambda i: (i, 0, 0)),
            pl.BlockSpec((BLOCK_N, h), lambda i: (i, 0)),
            pl.BlockSpec((BLOCK_N, 3 * d), lambda i: (i, 0)),
            pl.BlockSpec((3 * d, h), lambda i: (0, 0)),
            pl.BlockSpec((h, h), lambda i: (0, 0)),
            pl.BlockSpec((1, h), lambda i: (0, 0)),
            pl.BlockSpec((1, h), lambda i: (0, 0)),
            pl.BlockSpec((1, h), lambda i: (0, 0)),
        ],
        out_specs=pl.BlockSpec((BLOCK_N, l, h), lambda i: (i, 0, 0)),
        out_shape=jax.ShapeDtypeStruct((n, l, h), jnp.float32),
    )(memory, idx, x, w_in_t, w_h_t, bias, gamma, beta)
